# Initial kernel scaffold; baseline (speedup 1.0000x reference)
import jax
import jax.numpy as jnp
from jax.experimental import pallas as pl


def kernel(brics, function_group, pharmacophore, W1_brics, b1_brics, W2_brics, b2_brics, W1_fg, b1_fg, W2_fg, b2_fg, W1_ph, b1_ph, W2_ph, b2_ph, Wg, bwg, We, be, Wf, bf):
    raise NotImplementedError("write your pallas kernel here")


# speedup vs baseline: 3.1657x; 3.1657x over previous
import functools

import jax
import jax.numpy as jnp
from jax.experimental import pallas as pl
from jax.experimental.pallas import tpu as pltpu

H = 300
THR = 0.6
_ROWS = 1000


def _body(xb_ref, xf_ref, xp_ref,
          w1b_ref, w2b_ref, w1f_ref, w2f_ref, w1p_ref, w2p_ref,
          wg0_ref, wg1_ref, wg2_ref, we_ref, wf0_ref, wf1_ref,
          b1b_ref, b2b_ref, b1f_ref, b2f_ref, b1p_ref, b2p_ref,
          bwg_ref, be_ref, bfo_ref,
          out_ref):
    f32 = jnp.float32
    bf16 = jnp.bfloat16

    def enc(x_ref, w1_ref, b1_ref, w2_ref, b2_ref):
        h = jnp.dot(x_ref[...], w1_ref[...], preferred_element_type=f32)
        h = jnp.maximum(h + b1_ref[...], 0.0).astype(bf16)
        return jnp.dot(h, w2_ref[...], preferred_element_type=f32) + b2_ref[...]

    eb = enc(xb_ref, w1b_ref, b1b_ref, w2b_ref, b2b_ref)
    ef = enc(xf_ref, w1f_ref, b1f_ref, w2f_ref, b2f_ref)
    ep = enc(xp_ref, w1p_ref, b1p_ref, w2p_ref, b2p_ref)

    def normed(e):
        n = jnp.maximum(jnp.sqrt(jnp.sum(e * e, axis=1, keepdims=True)), 1e-12)
        return e / n

    fb, ff, fp = normed(eb), normed(ef), normed(ep)

    def pair(fi, fj, ei, ej):
        s = jnp.sum(fi * fj, axis=1, keepdims=True)
        hm = (fi * fj > THR).astype(f32)
        cf = (ei + ej) * 0.5 * hm
        pm = (s > THR).astype(f32)
        return s, cf, pm

    s01, cf01, pm01 = pair(fb, ff, eb, ef)
    s02, cf02, pm02 = pair(fb, fp, eb, ep)
    s12, cf12, pm12 = pair(ff, fp, ef, ep)

    ex01 = jnp.exp(s01) * pm01
    ex02 = jnp.exp(s02) * pm02
    ex12 = jnp.exp(s12) * pm12
    den = ex01 + ex02 + ex12
    safe = jnp.maximum(den, 1e-30)
    haspair = (pm01 + pm02 + pm12) > 0.0
    w01 = jnp.where(den > 0, ex01 / safe, 0.0)
    w02 = jnp.where(den > 0, ex02 / safe, 0.0)
    w12 = jnp.where(den > 0, ex12 / safe, 0.0)
    weighted = cf01 * w01 + cf02 * w02 + cf12 * w12
    mean_fps = (eb + ef + ep) * (1.0 / 3.0)
    common = jnp.where(haspair, weighted, mean_fps)

    eb16, ef16, ep16 = eb.astype(bf16), ef.astype(bf16), ep.astype(bf16)
    logits = (jnp.dot(eb16, wg0_ref[...], preferred_element_type=f32)
              + jnp.dot(ef16, wg1_ref[...], preferred_element_type=f32)
              + jnp.dot(ep16, wg2_ref[...], preferred_element_type=f32)
              + bwg_ref[0:1, 0:3])
    m = jnp.max(logits, axis=1, keepdims=True)
    el = jnp.exp(logits - m)
    fpw = el / jnp.sum(el, axis=1, keepdims=True)
    wfs = eb * fpw[:, 0:1] + ef * fpw[:, 1:2] + ep * fpw[:, 2:3]

    enh_in = jnp.dot(common.astype(bf16), we_ref[...], preferred_element_type=f32)
    enh = jax.nn.sigmoid(enh_in + be_ref[...])
    enhanced = common * enh

    out_ref[...] = (jnp.dot(wfs.astype(bf16), wf0_ref[...], preferred_element_type=f32)
                    + jnp.dot(enhanced.astype(bf16), wf1_ref[...], preferred_element_type=f32)
                    + bfo_ref[...])


@functools.partial(jax.jit, static_argnames=())
def kernel(brics, function_group, pharmacophore,
           W1_brics, b1_brics, W2_brics, b2_brics,
           W1_fg, b1_fg, W2_fg, b2_fg,
           W1_ph, b1_ph, W2_ph, b2_ph,
           Wg, bwg, We, be, Wf, bf):
    B = brics.shape[0]
    rows = _ROWS if B % _ROWS == 0 else (8 if B % 8 == 0 else 1)
    bf16 = jnp.bfloat16

    xb = brics.astype(bf16)
    xf = function_group.astype(bf16)
    xp = pharmacophore.astype(bf16)
    weights = [W1_brics.astype(bf16), W2_brics.astype(bf16),
               W1_fg.astype(bf16), W2_fg.astype(bf16),
               W1_ph.astype(bf16), W2_ph.astype(bf16),
               Wg[0:H].astype(bf16), Wg[H:2 * H].astype(bf16),
               Wg[2 * H:3 * H].astype(bf16),
               We.astype(bf16), Wf[0:H].astype(bf16), Wf[H:2 * H].astype(bf16)]
    bwg_pad = jnp.zeros((8, 128), jnp.float32).at[0, 0:3].set(bwg)
    biases = [b1_brics.reshape(1, H), b2_brics.reshape(1, H),
              b1_fg.reshape(1, H), b2_fg.reshape(1, H),
              b1_ph.reshape(1, H), b2_ph.reshape(1, H),
              bwg_pad, be.reshape(1, H), bf.reshape(1, H)]

    row_spec = pl.BlockSpec((rows, H), lambda i: (i, 0))
    full = lambda a: pl.BlockSpec(a.shape, lambda i: (0,) * a.ndim)

    return pl.pallas_call(
        _body,
        grid=(B // rows,),
        in_specs=[row_spec, row_spec, row_spec] + [full(w) for w in weights]
                 + [full(b) for b in biases],
        out_specs=row_spec,
        out_shape=jax.ShapeDtypeStruct((B, H), jnp.float32),
        compiler_params=pltpu.CompilerParams(
            dimension_semantics=("arbitrary",)),
    )(xb, xf, xp, *weights, *biases)


# speedup vs baseline: 3.4643x; 1.0943x over previous
import jax
import jax.numpy as jnp
from jax.experimental import pallas as pl
from jax.experimental.pallas import tpu as pltpu

H = 300
THR = 0.6
_ROWS = 1000


def _body(xb_ref, xf_ref, xp_ref,
          w1b_ref, w2b_ref, w1f_ref, w2f_ref, w1p_ref, w2p_ref,
          we_ref, wf0_ref, wf1_ref,
          b1b_ref, b2b_ref, b1f_ref, b2f_ref, b1p_ref, b2p_ref,
          blog_ref, be_ref, bfo_ref,
          out_ref):
    f32 = jnp.float32
    bf16 = jnp.bfloat16

    def enc(x_ref, w1_ref, b1_ref, w2_ref, b2_ref):
        h = jnp.dot(x_ref[...].astype(bf16), w1_ref[...],
                    preferred_element_type=f32)
        h = jnp.maximum(h + b1_ref[...], 0.0).astype(bf16)
        full = jnp.dot(h, w2_ref[...], preferred_element_type=f32)
        return full[:, 0:H] + b2_ref[...], full[:, H:H + 3]

    eb, lgb = enc(xb_ref, w1b_ref, b1b_ref, w2b_ref, b2b_ref)
    ef, lgf = enc(xf_ref, w1f_ref, b1f_ref, w2f_ref, b2f_ref)
    ep, lgp = enc(xp_ref, w1p_ref, b1p_ref, w2p_ref, b2p_ref)

    def rownorm(e):
        return jnp.maximum(
            jnp.sqrt(jnp.sum(e * e, axis=1, keepdims=True)), 1e-12)

    nb, nf, np_ = rownorm(eb), rownorm(ef), rownorm(ep)

    prod01 = eb * ef
    prod02 = eb * ep
    prod12 = ef * ep
    s01 = jnp.sum(prod01, axis=1, keepdims=True) / (nb * nf)
    s02 = jnp.sum(prod02, axis=1, keepdims=True) / (nb * np_)
    s12 = jnp.sum(prod12, axis=1, keepdims=True) / (nf * np_)

    pm01 = (s01 > THR).astype(f32)
    pm02 = (s02 > THR).astype(f32)
    pm12 = (s12 > THR).astype(f32)
    ex01 = jnp.exp(s01) * pm01
    ex02 = jnp.exp(s02) * pm02
    ex12 = jnp.exp(s12) * pm12
    den = ex01 + ex02 + ex12
    safe = jnp.maximum(den, 1e-30)
    haspair = (pm01 + pm02 + pm12) > 0.0
    a01 = jnp.where(den > 0, 0.5 * ex01 / safe, 0.0)
    a02 = jnp.where(den > 0, 0.5 * ex02 / safe, 0.0)
    a12 = jnp.where(den > 0, 0.5 * ex12 / safe, 0.0)
    weighted = ((eb + ef) * jnp.where(prod01 > THR * (nb * nf), a01, 0.0)
                + (eb + ep) * jnp.where(prod02 > THR * (nb * np_), a02, 0.0)
                + (ef + ep) * jnp.where(prod12 > THR * (nf * np_), a12, 0.0))
    mean_fps = (eb + ef + ep) * (1.0 / 3.0)
    common = jnp.where(haspair, weighted, mean_fps)

    logits = lgb + lgf + lgp + blog_ref[0:1, 0:3]
    m = jnp.max(logits, axis=1, keepdims=True)
    el = jnp.exp(logits - m)
    fpw = el / jnp.sum(el, axis=1, keepdims=True)
    wfs = eb * fpw[:, 0:1] + ef * fpw[:, 1:2] + ep * fpw[:, 2:3]

    enh_in = jnp.dot(common.astype(bf16), we_ref[...], preferred_element_type=f32)
    enh = jax.nn.sigmoid(enh_in + be_ref[...])
    enhanced = common * enh

    out_ref[...] = (jnp.dot(wfs.astype(bf16), wf0_ref[...], preferred_element_type=f32)
                    + jnp.dot(enhanced.astype(bf16), wf1_ref[...], preferred_element_type=f32)
                    + bfo_ref[...])


@jax.jit
def kernel(brics, function_group, pharmacophore,
           W1_brics, b1_brics, W2_brics, b2_brics,
           W1_fg, b1_fg, W2_fg, b2_fg,
           W1_ph, b1_ph, W2_ph, b2_ph,
           Wg, bwg, We, be, Wf, bf):
    B = brics.shape[0]
    rows = _ROWS if B % _ROWS == 0 else (8 if B % 8 == 0 else 1)
    bf16 = jnp.bfloat16

    wg0, wg1, wg2 = Wg[0:H], Wg[H:2 * H], Wg[2 * H:3 * H]
    w2b = jnp.concatenate([W2_brics, W2_brics @ wg0], axis=1).astype(bf16)
    w2f = jnp.concatenate([W2_fg, W2_fg @ wg1], axis=1).astype(bf16)
    w2p = jnp.concatenate([W2_ph, W2_ph @ wg2], axis=1).astype(bf16)
    blog = bwg + b2_brics @ wg0 + b2_fg @ wg1 + b2_ph @ wg2
    blog_pad = jnp.zeros((8, 128), jnp.float32).at[0, 0:3].set(blog)

    weights = [W1_brics.astype(bf16), w2b,
               W1_fg.astype(bf16), w2f,
               W1_ph.astype(bf16), w2p,
               We.astype(bf16), Wf[0:H].astype(bf16), Wf[H:2 * H].astype(bf16)]
    biases = [b1_brics.reshape(1, H), b2_brics.reshape(1, H),
              b1_fg.reshape(1, H), b2_fg.reshape(1, H),
              b1_ph.reshape(1, H), b2_ph.reshape(1, H),
              blog_pad, be.reshape(1, H), bf.reshape(1, H)]

    row_spec = pl.BlockSpec((rows, H), lambda i: (i, 0))
    full = lambda a: pl.BlockSpec(a.shape, lambda i: (0,) * a.ndim)

    return pl.pallas_call(
        _body,
        grid=(B // rows,),
        in_specs=[row_spec, row_spec, row_spec] + [full(w) for w in weights]
                 + [full(b) for b in biases],
        out_specs=row_spec,
        out_shape=jax.ShapeDtypeStruct((B, H), jnp.float32),
        compiler_params=pltpu.CompilerParams(
            dimension_semantics=("arbitrary",)),
    )(brics, function_group, pharmacophore, *weights, *biases)


# speedup vs baseline: 3.6597x; 1.0564x over previous
import jax
import jax.numpy as jnp
from jax.experimental import pallas as pl
from jax.experimental.pallas import tpu as pltpu

H = 300
THR = 0.6
_ROWS = 2000


def _body(xb_ref, xf_ref, xp_ref,
          w1b_ref, w2b_ref, w1f_ref, w2f_ref, w1p_ref, w2p_ref,
          we_ref, wf0_ref, wf1_ref,
          b1b_ref, b2b_ref, b1f_ref, b2f_ref, b1p_ref, b2p_ref,
          blog_ref, be_ref, bfo_ref,
          out_ref):
    f32 = jnp.float32
    bf16 = jnp.bfloat16

    def enc(x_ref, w1_ref, b1_ref, w2_ref, b2_ref):
        h = jnp.dot(x_ref[...].astype(bf16), w1_ref[...],
                    preferred_element_type=f32)
        h = jnp.maximum(h + b1_ref[...], 0.0).astype(bf16)
        full = jnp.dot(h, w2_ref[...], preferred_element_type=f32)
        return full[:, 0:H] + b2_ref[...], full[:, H:H + 3]

    eb, lgb = enc(xb_ref, w1b_ref, b1b_ref, w2b_ref, b2b_ref)
    ef, lgf = enc(xf_ref, w1f_ref, b1f_ref, w2f_ref, b2f_ref)
    ep, lgp = enc(xp_ref, w1p_ref, b1p_ref, w2p_ref, b2p_ref)

    def rownorm(e):
        return jnp.maximum(
            jnp.sqrt(jnp.sum(e * e, axis=1, keepdims=True)), 1e-12)

    nb, nf, np_ = rownorm(eb), rownorm(ef), rownorm(ep)

    prod01 = eb * ef
    prod02 = eb * ep
    prod12 = ef * ep
    s01 = jnp.sum(prod01, axis=1, keepdims=True) / (nb * nf)
    s02 = jnp.sum(prod02, axis=1, keepdims=True) / (nb * np_)
    s12 = jnp.sum(prod12, axis=1, keepdims=True) / (nf * np_)

    pm01 = (s01 > THR).astype(f32)
    pm02 = (s02 > THR).astype(f32)
    pm12 = (s12 > THR).astype(f32)
    ex01 = jnp.exp(s01) * pm01
    ex02 = jnp.exp(s02) * pm02
    ex12 = jnp.exp(s12) * pm12
    den = ex01 + ex02 + ex12
    safe = jnp.maximum(den, 1e-30)
    haspair = (pm01 + pm02 + pm12) > 0.0
    a01 = jnp.where(den > 0, 0.5 * ex01 / safe, 0.0)
    a02 = jnp.where(den > 0, 0.5 * ex02 / safe, 0.0)
    a12 = jnp.where(den > 0, 0.5 * ex12 / safe, 0.0)
    weighted = ((eb + ef) * jnp.where(prod01 > THR * (nb * nf), a01, 0.0)
                + (eb + ep) * jnp.where(prod02 > THR * (nb * np_), a02, 0.0)
                + (ef + ep) * jnp.where(prod12 > THR * (nf * np_), a12, 0.0))
    mean_fps = (eb + ef + ep) * (1.0 / 3.0)
    common = jnp.where(haspair, weighted, mean_fps)

    logits = lgb + lgf + lgp + blog_ref[0:1, 0:3]
    m = jnp.max(logits, axis=1, keepdims=True)
    el = jnp.exp(logits - m)
    fpw = el / jnp.sum(el, axis=1, keepdims=True)
    wfs = eb * fpw[:, 0:1] + ef * fpw[:, 1:2] + ep * fpw[:, 2:3]

    enh_in = jnp.dot(common.astype(bf16), we_ref[...], preferred_element_type=f32)
    enh = jax.nn.sigmoid(enh_in + be_ref[...])
    enhanced = common * enh

    out_ref[...] = (jnp.dot(wfs.astype(bf16), wf0_ref[...], preferred_element_type=f32)
                    + jnp.dot(enhanced.astype(bf16), wf1_ref[...], preferred_element_type=f32)
                    + bfo_ref[...])


@jax.jit
def kernel(brics, function_group, pharmacophore,
           W1_brics, b1_brics, W2_brics, b2_brics,
           W1_fg, b1_fg, W2_fg, b2_fg,
           W1_ph, b1_ph, W2_ph, b2_ph,
           Wg, bwg, We, be, Wf, bf):
    B = brics.shape[0]
    rows = _ROWS if B % _ROWS == 0 else (8 if B % 8 == 0 else 1)
    bf16 = jnp.bfloat16

    wg0, wg1, wg2 = Wg[0:H], Wg[H:2 * H], Wg[2 * H:3 * H]
    w2b = jnp.concatenate([W2_brics, W2_brics @ wg0], axis=1).astype(bf16)
    w2f = jnp.concatenate([W2_fg, W2_fg @ wg1], axis=1).astype(bf16)
    w2p = jnp.concatenate([W2_ph, W2_ph @ wg2], axis=1).astype(bf16)
    blog = bwg + b2_brics @ wg0 + b2_fg @ wg1 + b2_ph @ wg2
    blog_pad = jnp.zeros((8, 128), jnp.float32).at[0, 0:3].set(blog)

    weights = [W1_brics.astype(bf16), w2b,
               W1_fg.astype(bf16), w2f,
               W1_ph.astype(bf16), w2p,
               We.astype(bf16), Wf[0:H].astype(bf16), Wf[H:2 * H].astype(bf16)]
    biases = [b1_brics.reshape(1, H), b2_brics.reshape(1, H),
              b1_fg.reshape(1, H), b2_fg.reshape(1, H),
              b1_ph.reshape(1, H), b2_ph.reshape(1, H),
              blog_pad, be.reshape(1, H), bf.reshape(1, H)]

    row_spec = pl.BlockSpec((rows, H), lambda i: (i, 0))
    full = lambda a: pl.BlockSpec(a.shape, lambda i: (0,) * a.ndim)

    return pl.pallas_call(
        _body,
        grid=(B // rows,),
        in_specs=[row_spec, row_spec, row_spec] + [full(w) for w in weights]
                 + [full(b) for b in biases],
        out_specs=row_spec,
        out_shape=jax.ShapeDtypeStruct((B, H), jnp.float32),
        compiler_params=pltpu.CompilerParams(
            dimension_semantics=("arbitrary",)),
    )(brics, function_group, pharmacophore, *weights, *biases)
